# baseline (speedup 1.0000x reference)
import functools

import jax
import jax.numpy as jnp
from jax import lax
from jax.experimental import pallas as pl
from jax.experimental.pallas import tpu as pltpu
from jax.experimental.pallas import tpu_sc as plsc

D = 32
H = 128
T = 9
L_SEQ = 50

_NC = 2
_NS = 16
_NW = _NC * _NS
_CHUNK = 128


def _sc_gather(table, idx3, n_rows):
    n_chunks = idx3.shape[1]
    mesh = plsc.VectorSubcoreMesh(core_axis_name="c", subcore_axis_name="s")

    @functools.partial(
        pl.kernel,
        mesh=mesh,
        out_type=jax.ShapeDtypeStruct((n_rows, D), jnp.float32),
        compiler_params=pltpu.CompilerParams(use_tc_tiling_on_sc=False),
        scratch_types=[
            pltpu.VMEM((n_chunks, _CHUNK), jnp.int32),
            pltpu.VMEM((_CHUNK, D), jnp.float32),
            pltpu.SemaphoreType.DMA,
        ],
    )
    def k(table_hbm, idx_hbm, out_hbm, idx_v, rows_v, sem):
        wid = lax.axis_index("s") * _NC + lax.axis_index("c")
        pltpu.sync_copy(idx_hbm.at[wid], idx_v)

        def body(j, carry):
            pltpu.async_copy(table_hbm.at[idx_v.at[j]], rows_v, sem).wait()
            base = (wid * n_chunks + j) * _CHUNK
            pltpu.sync_copy(rows_v, out_hbm.at[pl.ds(base, _CHUNK)])
            return carry

        lax.fori_loop(0, n_chunks, body, 0)

    return k(table, idx3)


def _lstm_body(x_ref, wcat_ref, b_ref, wfc_ref, bfc_ref, out_ref):
    wcat = wcat_ref[...]
    b = b_ref[...]
    wfc = wfc_ref[...]
    bfc = bfc_ref[...]
    bt = x_ref.shape[0]

    def step(t, carry):
        h, c = carry
        x_t = x_ref[:, t, :]
        inp = jnp.concatenate([x_t, h], axis=1)
        gates = jnp.dot(inp, wcat, preferred_element_type=jnp.float32) + b
        ig = jax.nn.sigmoid(gates[:, 0:H])
        fg = jax.nn.sigmoid(gates[:, H:2 * H])
        gg = jnp.tanh(gates[:, 2 * H:3 * H])
        og = jax.nn.sigmoid(gates[:, 3 * H:4 * H])
        c = fg * c + ig * gg
        h = og * jnp.tanh(c)
        logits = jnp.dot(h, wfc, preferred_element_type=jnp.float32) + bfc
        m = jnp.max(logits, axis=-1, keepdims=True)
        lse = m + jnp.log(jnp.sum(jnp.exp(logits - m), axis=-1, keepdims=True))
        out_ref[:, t, :] = logits - lse
        return (h, c)

    init = (jnp.zeros((bt, H), jnp.float32), jnp.zeros((bt, H), jnp.float32))
    lax.fori_loop(0, L_SEQ, step, init)


def _lstm_fc(x, wcat, bias, wfc, bfc, bt=512):
    B = x.shape[0]
    return pl.pallas_call(
        _lstm_body,
        grid=(B // bt,),
        in_specs=[
            pl.BlockSpec((bt, L_SEQ, D), lambda i: (i, 0, 0)),
            pl.BlockSpec((D + H, 4 * H), lambda i: (0, 0)),
            pl.BlockSpec((1, 4 * H), lambda i: (0, 0)),
            pl.BlockSpec((H, T), lambda i: (0, 0)),
            pl.BlockSpec((1, T), lambda i: (0, 0)),
        ],
        out_specs=pl.BlockSpec((bt, L_SEQ, T), lambda i: (i, 0, 0)),
        out_shape=jax.ShapeDtypeStruct((B, L_SEQ, T), jnp.float32),
    )(x, wcat, bias, wfc, bfc)


def kernel(sentences, labels, emb_table, W_ih, W_hh, b_ih, b_hh, W_fc, b_fc):
    B, L = sentences.shape
    n_rows = B * L
    idx = sentences.reshape(-1).astype(jnp.int32)
    idx3 = idx.reshape(_NW, -1, _CHUNK)
    x = _sc_gather(emb_table, idx3, n_rows).reshape(B, L, D)

    wcat = jnp.concatenate([W_ih, W_hh], axis=1).T
    bias = (b_ih + b_hh).reshape(1, 4 * H)
    wfc = W_fc.T
    bfc = b_fc.reshape(1, T)

    return _lstm_fc(x, wcat, bias, wfc, bfc)


# speedup vs baseline: 1.0517x; 1.0517x over previous
import functools

import jax
import jax.numpy as jnp
from jax import lax
from jax.experimental import pallas as pl
from jax.experimental.pallas import tpu as pltpu
from jax.experimental.pallas import tpu_sc as plsc

D = 32
H = 128
T = 9
L_SEQ = 50

_NC = 2
_NS = 16
_NW = _NC * _NS
_CHUNK = 128


def _sc_gather(table, idx3, n_rows):
    n_chunks = idx3.shape[1]
    mesh = plsc.VectorSubcoreMesh(core_axis_name="c", subcore_axis_name="s")

    @functools.partial(
        pl.kernel,
        mesh=mesh,
        out_type=jax.ShapeDtypeStruct((n_rows, D), jnp.float32),
        compiler_params=pltpu.CompilerParams(use_tc_tiling_on_sc=False),
        scratch_types=[
            pltpu.VMEM((n_chunks, _CHUNK), jnp.int32),
            pltpu.VMEM((_CHUNK, D), jnp.float32),
            pltpu.SemaphoreType.DMA,
        ],
    )
    def k(table_hbm, idx_hbm, out_hbm, idx_v, rows_v, sem):
        wid = lax.axis_index("s") * _NC + lax.axis_index("c")
        pltpu.sync_copy(idx_hbm.at[wid], idx_v)

        def body(j, carry):
            pltpu.async_copy(table_hbm.at[idx_v.at[j]], rows_v, sem).wait()
            base = (wid * n_chunks + j) * _CHUNK
            pltpu.sync_copy(rows_v, out_hbm.at[pl.ds(base, _CHUNK)])
            return carry

        lax.fori_loop(0, n_chunks, body, 0)

    return k(table, idx3)


def _sigmoid(x):
    return 0.5 * jnp.tanh(0.5 * x) + 0.5


def _lstm_body(x_ref, wih_ref, whh_ref, b_ref, wfc_ref, bfc_ref, out_ref,
               hs_ref):
    wih = wih_ref[...]
    whh = whh_ref[...]
    b = b_ref[...]
    bt = x_ref.shape[1]

    def step(t, carry):
        h, c = carry
        x_t = x_ref[t]
        gates = (jnp.dot(x_t, wih, preferred_element_type=jnp.float32)
                 + jnp.dot(h, whh, preferred_element_type=jnp.float32) + b)
        ig = _sigmoid(gates[:, 0:H])
        fg = _sigmoid(gates[:, H:2 * H])
        gg = jnp.tanh(gates[:, 2 * H:3 * H])
        og = _sigmoid(gates[:, 3 * H:4 * H])
        c = fg * c + ig * gg
        h = og * jnp.tanh(c)
        hs_ref[pl.ds(t * bt, bt), :] = h
        return (h, c)

    init = (jnp.zeros((bt, H), jnp.float32), jnp.zeros((bt, H), jnp.float32))
    lax.fori_loop(0, L_SEQ, step, init)

    logits = (jnp.dot(hs_ref[...], wfc_ref[...],
                      preferred_element_type=jnp.float32) + bfc_ref[...])
    m = jnp.max(logits, axis=-1, keepdims=True)
    lse = m + jnp.log(jnp.sum(jnp.exp(logits - m), axis=-1, keepdims=True))
    out_ref[...] = (logits - lse).reshape(L_SEQ, bt, T)


def _lstm_fc(x, wih, whh, bias, wfc, bfc, bt=256):
    B = x.shape[1]
    return pl.pallas_call(
        _lstm_body,
        grid=(B // bt,),
        in_specs=[
            pl.BlockSpec((L_SEQ, bt, D), lambda i: (0, i, 0)),
            pl.BlockSpec((D, 4 * H), lambda i: (0, 0)),
            pl.BlockSpec((H, 4 * H), lambda i: (0, 0)),
            pl.BlockSpec((1, 4 * H), lambda i: (0, 0)),
            pl.BlockSpec((H, T), lambda i: (0, 0)),
            pl.BlockSpec((1, T), lambda i: (0, 0)),
        ],
        out_specs=pl.BlockSpec((L_SEQ, bt, T), lambda i: (0, i, 0)),
        out_shape=jax.ShapeDtypeStruct((L_SEQ, B, T), jnp.float32),
        scratch_shapes=[pltpu.VMEM((L_SEQ * bt, H), jnp.float32)],
    )(x, wih, whh, bias, wfc, bfc)


def kernel(sentences, labels, emb_table, W_ih, W_hh, b_ih, b_hh, W_fc, b_fc):
    B, L = sentences.shape
    n_rows = B * L
    idx = jnp.swapaxes(sentences, 0, 1).reshape(-1).astype(jnp.int32)
    idx3 = idx.reshape(_NW, -1, _CHUNK)
    x = _sc_gather(emb_table, idx3, n_rows).reshape(L, B, D)

    wih = W_ih.T
    whh = W_hh.T
    bias = (b_ih + b_hh).reshape(1, 4 * H)
    wfc = W_fc.T
    bfc = b_fc.reshape(1, T)

    out_tm = _lstm_fc(x, wih, whh, bias, wfc, bfc)
    return jnp.swapaxes(out_tm, 0, 1)


# speedup vs baseline: 1.0640x; 1.0116x over previous
import functools

import jax
import jax.numpy as jnp
from jax import lax
from jax.experimental import pallas as pl
from jax.experimental.pallas import tpu as pltpu
from jax.experimental.pallas import tpu_sc as plsc

D = 32
H = 128
T = 9
L_SEQ = 50

_NC = 2
_NS = 16
_NW = _NC * _NS
_CHUNK = 128


def _sc_gather(table, idx3, n_rows):
    n_chunks = idx3.shape[1]
    mesh = plsc.VectorSubcoreMesh(core_axis_name="c", subcore_axis_name="s")

    @functools.partial(
        pl.kernel,
        mesh=mesh,
        out_type=jax.ShapeDtypeStruct((n_rows, D), jnp.float32),
        compiler_params=pltpu.CompilerParams(use_tc_tiling_on_sc=False),
        scratch_types=[
            pltpu.VMEM((n_chunks, _CHUNK), jnp.int32),
            pltpu.VMEM((_CHUNK, D), jnp.float32),
            pltpu.SemaphoreType.DMA,
        ],
    )
    def k(table_hbm, idx_hbm, out_hbm, idx_v, rows_v, sem):
        wid = lax.axis_index("s") * _NC + lax.axis_index("c")
        pltpu.sync_copy(idx_hbm.at[wid], idx_v)

        def body(j, carry):
            pltpu.async_copy(table_hbm.at[idx_v.at[j]], rows_v, sem).wait()
            base = (wid * n_chunks + j) * _CHUNK
            pltpu.sync_copy(rows_v, out_hbm.at[pl.ds(base, _CHUNK)])
            return carry

        lax.fori_loop(0, n_chunks, body, 0)

    return k(table, idx3)


def _sigmoid(x):
    return 0.5 * jnp.tanh(0.5 * x) + 0.5


def _lstm_body(x_ref, wih_ref, whh_ref, b_ref, wfc_ref, bfc_ref, out_ref):
    wih = wih_ref[...]
    whh = whh_ref[...]
    b = b_ref[...]
    wfc = wfc_ref[...]
    bfc = bfc_ref[...]
    bt = x_ref.shape[1]

    def step(t, carry):
        h, c = carry
        x_t = x_ref[t]
        gates = (jnp.dot(x_t, wih, preferred_element_type=jnp.float32)
                 + jnp.dot(h, whh, preferred_element_type=jnp.float32) + b)
        ig = _sigmoid(gates[:, 0:H])
        fg = _sigmoid(gates[:, H:2 * H])
        gg = jnp.tanh(gates[:, 2 * H:3 * H])
        og = _sigmoid(gates[:, 3 * H:4 * H])
        c = fg * c + ig * gg
        h = og * jnp.tanh(c)
        logits = jnp.dot(h, wfc, preferred_element_type=jnp.float32) + bfc
        m = jnp.max(logits, axis=-1, keepdims=True)
        lse = m + jnp.log(jnp.sum(jnp.exp(logits - m), axis=-1, keepdims=True))
        out_ref[t] = logits - lse
        return (h, c)

    init = (jnp.zeros((bt, H), jnp.float32), jnp.zeros((bt, H), jnp.float32))
    lax.fori_loop(0, L_SEQ, step, init)


def _lstm_fc(x, wih, whh, bias, wfc, bfc, bt=512):
    B = x.shape[1]
    return pl.pallas_call(
        _lstm_body,
        grid=(B // bt,),
        in_specs=[
            pl.BlockSpec((L_SEQ, bt, D), lambda i: (0, i, 0)),
            pl.BlockSpec((D, 4 * H), lambda i: (0, 0)),
            pl.BlockSpec((H, 4 * H), lambda i: (0, 0)),
            pl.BlockSpec((1, 4 * H), lambda i: (0, 0)),
            pl.BlockSpec((H, T), lambda i: (0, 0)),
            pl.BlockSpec((1, T), lambda i: (0, 0)),
        ],
        out_specs=pl.BlockSpec((L_SEQ, bt, T), lambda i: (0, i, 0)),
        out_shape=jax.ShapeDtypeStruct((L_SEQ, B, T), jnp.float32),
    )(x, wih, whh, bias, wfc, bfc)


def kernel(sentences, labels, emb_table, W_ih, W_hh, b_ih, b_hh, W_fc, b_fc):
    B, L = sentences.shape
    n_rows = B * L
    idx = jnp.swapaxes(sentences, 0, 1).reshape(-1).astype(jnp.int32)
    idx3 = idx.reshape(_NW, -1, _CHUNK)
    x = _sc_gather(emb_table, idx3, n_rows).reshape(L, B, D)

    wih = W_ih.T
    whh = W_hh.T
    bias = (b_ih + b_hh).reshape(1, 4 * H)
    wfc = W_fc.T
    bfc = b_fc.reshape(1, T)

    out_tm = _lstm_fc(x, wih, whh, bias, wfc, bfc)
    return jnp.swapaxes(out_tm, 0, 1)


# speedup vs baseline: 1.2072x; 1.1346x over previous
import functools

import jax
import jax.numpy as jnp
from jax import lax
from jax.experimental import pallas as pl
from jax.experimental.pallas import tpu as pltpu
from jax.experimental.pallas import tpu_sc as plsc

D = 32
H = 128
T = 9
L_SEQ = 50

_NC = 2
_NS = 16
_NW = _NC * _NS
_CHUNK = 128


def _sc_gather(table, idx3, n_rows):
    n_chunks = idx3.shape[1]
    mesh = plsc.VectorSubcoreMesh(core_axis_name="c", subcore_axis_name="s")

    @functools.partial(
        pl.kernel,
        mesh=mesh,
        out_type=jax.ShapeDtypeStruct((n_rows, D), jnp.float32),
        compiler_params=pltpu.CompilerParams(use_tc_tiling_on_sc=False),
        scratch_types=[
            pltpu.VMEM((n_chunks, _CHUNK), jnp.int32),
            pltpu.VMEM((_CHUNK, D), jnp.float32),
            pltpu.SemaphoreType.DMA,
        ],
    )
    def k(table_hbm, idx_hbm, out_hbm, idx_v, rows_v, sem):
        wid = lax.axis_index("s") * _NC + lax.axis_index("c")
        pltpu.sync_copy(idx_hbm.at[wid], idx_v)

        def body(j, carry):
            pltpu.async_copy(table_hbm.at[idx_v.at[j]], rows_v, sem).wait()
            base = (wid * n_chunks + j) * _CHUNK
            pltpu.sync_copy(rows_v, out_hbm.at[pl.ds(base, _CHUNK)])
            return carry

        lax.fori_loop(0, n_chunks, body, 0)

    return k(table, idx3)


def _sigmoid(x):
    return 0.5 * jnp.tanh(0.5 * x) + 0.5


def _lstm_body(x_ref, wih_ref, whh_ref, b_ref, wfc_ref, bfc_ref, out_ref,
               hs_ref):
    wih = wih_ref[...]
    whh = whh_ref[...]
    b = b_ref[...]
    bt = x_ref.shape[1]

    def step(t, carry):
        h, c = carry
        x_t = x_ref[t]
        gates = (jnp.dot(x_t, wih, preferred_element_type=jnp.float32)
                 + jnp.dot(h, whh, preferred_element_type=jnp.float32) + b)
        ig = _sigmoid(gates[:, 0:H])
        fg = _sigmoid(gates[:, H:2 * H])
        gg = jnp.tanh(gates[:, 2 * H:3 * H])
        og = _sigmoid(gates[:, 3 * H:4 * H])
        c = fg * c + ig * gg
        h = og * jnp.tanh(c)
        hs_ref[pl.ds(t * bt, bt), :] = h
        return (h, c)

    init = (jnp.zeros((bt, H), jnp.float32), jnp.zeros((bt, H), jnp.float32))
    lax.fori_loop(0, L_SEQ, step, init)

    wfc9 = wfc_ref[...]
    bfc9 = bfc_ref[...]
    for t in range(L_SEQ):
        h_t = hs_ref[pl.ds(t * bt, bt), :]
        lT = jax.lax.dot_general(wfc9, h_t, (((1,), (1,)), ((), ())),
                                 preferred_element_type=jnp.float32) + bfc9
        m = jnp.max(lT, axis=0, keepdims=True)
        e = jnp.exp(lT - m)
        lse = m + jnp.log(jnp.sum(e, axis=0, keepdims=True))
        out_ref[:, t, :] = lT - lse


def _lstm_fc(x, wih, whh, bias, wfc, bfc, bt=512):
    B = x.shape[1]
    return pl.pallas_call(
        _lstm_body,
        grid=(B // bt,),
        in_specs=[
            pl.BlockSpec((L_SEQ, bt, D), lambda i: (0, i, 0)),
            pl.BlockSpec((D, 4 * H), lambda i: (0, 0)),
            pl.BlockSpec((H, 4 * H), lambda i: (0, 0)),
            pl.BlockSpec((1, 4 * H), lambda i: (0, 0)),
            pl.BlockSpec((T, H), lambda i: (0, 0)),
            pl.BlockSpec((T, 1), lambda i: (0, 0)),
        ],
        out_specs=pl.BlockSpec((T, L_SEQ, bt), lambda i: (0, 0, i)),
        out_shape=jax.ShapeDtypeStruct((T, L_SEQ, B), jnp.float32),
        scratch_shapes=[pltpu.VMEM((L_SEQ * bt, H), jnp.float32)],
    )(x, wih, whh, bias, wfc, bfc)


def kernel(sentences, labels, emb_table, W_ih, W_hh, b_ih, b_hh, W_fc, b_fc):
    B, L = sentences.shape
    n_rows = B * L
    idx = jnp.swapaxes(sentences, 0, 1).reshape(-1).astype(jnp.int32)
    idx3 = idx.reshape(_NW, -1, _CHUNK)
    x = _sc_gather(emb_table, idx3, n_rows).reshape(L, B, D)

    wih = W_ih.T
    whh = W_hh.T
    bias = (b_ih + b_hh).reshape(1, 4 * H)
    bfc = b_fc.reshape(T, 1)

    out_t = _lstm_fc(x, wih, whh, bias, W_fc, bfc)
    return jnp.transpose(out_t, (2, 1, 0))
